# baseline (speedup 1.0000x reference)
import jax
import jax.numpy as jnp
from jax.experimental import pallas as pl
from jax.experimental.pallas import tpu as pltpu

_B = 64
_V = 1000000
_K = 10
_C = 32768
_NCH = (_V + _C - 1) // _C

_NEG = float("-inf")


def _topk_body(x_ref, o_ref, xs, R):
    j = pl.program_id(0)

    @pl.when(j == 0)
    def _():
        R[...] = jnp.full((_B, 128), _NEG, jnp.float32)

    col = jax.lax.broadcasted_iota(jnp.int32, (_B, _C), 1) + j * _C
    x = jnp.where(col < _V, x_ref[...], _NEG)

    Rv = R[...]
    m = jnp.max(x, axis=1, keepdims=True)

    @pl.when(jnp.any(m > Rv[:, 9:10]))
    def _():
        xs[...] = x

        def cond(c):
            Rc, mc = c
            return jnp.any(mc > Rc[:, 9:10])

        def body(c):
            Rc, mc = c
            mb = jnp.broadcast_to(mc, (_B, 128))
            ge = Rc >= mb
            Rsh = jnp.concatenate(
                [jnp.full((_B, 1), jnp.inf, jnp.float32), Rc[:, :-1]],
                axis=1)
            gesh = Rsh >= mb
            Rc = jnp.where(ge, Rc, jnp.where(gesh, mb, Rsh))
            xv = xs[...]
            eq = xv == mc
            am = jnp.min(jnp.where(eq, col, jnp.int32(2 ** 30)), axis=1,
                         keepdims=True)
            xn = jnp.where(col == am, _NEG, xv)
            xs[...] = xn
            mc = jnp.max(xn, axis=1, keepdims=True)
            return Rc, mc

        Rn, _ = jax.lax.while_loop(cond, body, (Rv, m))
        R[...] = Rn

    @pl.when(j == pl.num_programs(0) - 1)
    def _():
        o_ref[...] = R[...]


def _mlp_body(t_ref, w0_ref, b0_ref, w1_ref, b1_ref, w2_ref, b2_ref, o_ref):
    t = t_ref[...]
    h = jnp.maximum(
        jnp.dot(t, w0_ref[...], preferred_element_type=jnp.float32)
        + b0_ref[...], 0.0)
    h = jnp.maximum(
        jnp.dot(h, w1_ref[...], preferred_element_type=jnp.float32)
        + b1_ref[...], 0.0)
    tt = jnp.dot(h, w2_ref[...], preferred_element_type=jnp.float32) \
        + b2_ref[...]
    temp = jnp.clip(jnp.abs(tt), 1e-12, 1e12)
    o_ref[...] = 1.0 / temp


def _sum_body(x_ref, m_ref, it_ref, o_ref):
    j = pl.program_id(0)

    @pl.when(j == 0)
    def _():
        o_ref[...] = jnp.zeros_like(o_ref)

    col = jax.lax.broadcasted_iota(jnp.int32, (_B, _C), 1) + j * _C
    e = jnp.exp((x_ref[...] - m_ref[...]) * it_ref[...])
    e = jnp.where(col < _V, e, 0.0)
    o_ref[...] = o_ref[...] + jnp.sum(e, axis=1, keepdims=True)


def _scale_body(x_ref, m_ref, it_ref, s_ref, o_ref):
    inv_s = 1.0 / s_ref[...]
    o_ref[...] = jnp.exp((x_ref[...] - m_ref[...]) * it_ref[...]) * inv_s


def _pad2(w, rows, cols):
    return jnp.pad(w, ((0, rows - w.shape[0]), (0, cols - w.shape[1])))


def kernel(logits, W0, b0, W1, b1, W2, b2):
    f32 = jnp.float32
    seq = pltpu.CompilerParams(dimension_semantics=("arbitrary",))

    topk = pl.pallas_call(
        _topk_body,
        grid=(_NCH,),
        in_specs=[pl.BlockSpec((_B, _C), lambda j: (0, j))],
        out_specs=pl.BlockSpec((_B, 128), lambda j: (0, 0)),
        out_shape=jax.ShapeDtypeStruct((_B, 128), f32),
        scratch_shapes=[
            pltpu.VMEM((_B, _C), f32),
            pltpu.VMEM((_B, 128), f32),
        ],
        compiler_params=seq,
    )(logits)

    m = topk[:, :1]
    t10p = jnp.pad(topk[:, :_K], ((0, 0), (0, 128 - _K)))

    w0p = _pad2(W0, 128, 128)
    w1p = _pad2(W1, 128, 128)
    w2p = _pad2(W2, 128, 128)
    b0p = _pad2(b0[None, :], 1, 128)
    b1p = _pad2(b1[None, :], 1, 128)
    b2p = _pad2(b2[None, :], 1, 128)

    inv_t = pl.pallas_call(
        _mlp_body,
        out_shape=jax.ShapeDtypeStruct((_B, 128), f32),
    )(t10p, w0p, b0p, w1p, b1p, w2p, b2p)[:, :1]

    s = pl.pallas_call(
        _sum_body,
        grid=(_NCH,),
        in_specs=[
            pl.BlockSpec((_B, _C), lambda j: (0, j)),
            pl.BlockSpec((_B, 1), lambda j: (0, 0)),
            pl.BlockSpec((_B, 1), lambda j: (0, 0)),
        ],
        out_specs=pl.BlockSpec((_B, 128), lambda j: (0, 0)),
        out_shape=jax.ShapeDtypeStruct((_B, 128), f32),
        compiler_params=seq,
    )(logits, m, inv_t)[:, :1]

    probs = pl.pallas_call(
        _scale_body,
        grid=(_NCH,),
        in_specs=[
            pl.BlockSpec((_B, _C), lambda j: (0, j)),
            pl.BlockSpec((_B, 1), lambda j: (0, 0)),
            pl.BlockSpec((_B, 1), lambda j: (0, 0)),
            pl.BlockSpec((_B, 1), lambda j: (0, 0)),
        ],
        out_specs=pl.BlockSpec((_B, _C), lambda j: (0, j)),
        out_shape=jax.ShapeDtypeStruct((_B, _V), f32),
        compiler_params=seq,
    )(logits, m, inv_t, s)

    return probs


# speedup vs baseline: 1.0364x; 1.0364x over previous
import jax
import jax.numpy as jnp
from jax.experimental import pallas as pl
from jax.experimental.pallas import tpu as pltpu

_B = 64
_V = 1000000
_K = 10
_C = 16384
_NCH = (_V + _C - 1) // _C

_NEG = float("-inf")


def _topk_body(x_ref, o_ref, xs, R):
    j = pl.program_id(0)

    @pl.when(j == 0)
    def _():
        R[...] = jnp.full((_B, 128), _NEG, jnp.float32)

    col = jax.lax.broadcasted_iota(jnp.int32, (_B, _C), 1) + j * _C
    x = jnp.where(col < _V, x_ref[...], _NEG)

    Rv = R[...]
    m = jnp.max(x, axis=1, keepdims=True)

    @pl.when(jnp.any(m > Rv[:, 9:10]))
    def _():
        xs[...] = x

        def cond(c):
            Rc, mc = c
            return jnp.any(mc > Rc[:, 9:10])

        def body(c):
            Rc, mc = c
            mb = jnp.broadcast_to(mc, (_B, 128))
            ge = Rc >= mb
            Rsh = jnp.concatenate(
                [jnp.full((_B, 1), jnp.inf, jnp.float32), Rc[:, :-1]],
                axis=1)
            gesh = Rsh >= mb
            Rc = jnp.where(ge, Rc, jnp.where(gesh, mb, Rsh))
            xv = xs[...]
            eq = xv == mc
            am = jnp.min(jnp.where(eq, col, jnp.int32(2 ** 30)), axis=1,
                         keepdims=True)
            xn = jnp.where(col == am, _NEG, xv)
            xs[...] = xn
            mc = jnp.max(xn, axis=1, keepdims=True)
            return Rc, mc

        Rn, _ = jax.lax.while_loop(cond, body, (Rv, m))
        R[...] = Rn

    @pl.when(j == pl.num_programs(0) - 1)
    def _():
        o_ref[...] = R[...]


def _mlp_body(t_ref, w0_ref, b0_ref, w1_ref, b1_ref, w2_ref, b2_ref, o_ref):
    t = t_ref[...]
    h = jnp.maximum(
        jnp.dot(t, w0_ref[...], preferred_element_type=jnp.float32)
        + b0_ref[...], 0.0)
    h = jnp.maximum(
        jnp.dot(h, w1_ref[...], preferred_element_type=jnp.float32)
        + b1_ref[...], 0.0)
    tt = jnp.dot(h, w2_ref[...], preferred_element_type=jnp.float32) \
        + b2_ref[...]
    temp = jnp.clip(jnp.abs(tt), 1e-12, 1e12)
    o_ref[...] = 1.0 / temp


def _sum_body(x_ref, m_ref, it_ref, o_ref):
    j = pl.program_id(0)

    @pl.when(j == 0)
    def _():
        o_ref[...] = jnp.zeros_like(o_ref)

    col = jax.lax.broadcasted_iota(jnp.int32, (_B, _C), 1) + j * _C
    e = jnp.exp((x_ref[...] - m_ref[...]) * it_ref[...])
    e = jnp.where(col < _V, e, 0.0)
    o_ref[...] = o_ref[...] + jnp.sum(e, axis=1, keepdims=True)


def _scale_body(x_ref, m_ref, it_ref, s_ref, o_ref):
    inv_s = 1.0 / s_ref[...]
    o_ref[...] = jnp.exp((x_ref[...] - m_ref[...]) * it_ref[...]) * inv_s


def _pad2(w, rows, cols):
    return jnp.pad(w, ((0, rows - w.shape[0]), (0, cols - w.shape[1])))


def kernel(logits, W0, b0, W1, b1, W2, b2):
    f32 = jnp.float32
    seq = pltpu.CompilerParams(dimension_semantics=("arbitrary",))

    topk = pl.pallas_call(
        _topk_body,
        grid=(_NCH,),
        in_specs=[pl.BlockSpec((_B, _C), lambda j: (0, j))],
        out_specs=pl.BlockSpec((_B, 128), lambda j: (0, 0)),
        out_shape=jax.ShapeDtypeStruct((_B, 128), f32),
        scratch_shapes=[
            pltpu.VMEM((_B, _C), f32),
            pltpu.VMEM((_B, 128), f32),
        ],
        compiler_params=seq,
    )(logits)

    m = topk[:, :1]
    t10p = jnp.pad(topk[:, :_K], ((0, 0), (0, 128 - _K)))

    w0p = _pad2(W0, 128, 128)
    w1p = _pad2(W1, 128, 128)
    w2p = _pad2(W2, 128, 128)
    b0p = _pad2(b0[None, :], 1, 128)
    b1p = _pad2(b1[None, :], 1, 128)
    b2p = _pad2(b2[None, :], 1, 128)

    inv_t = pl.pallas_call(
        _mlp_body,
        out_shape=jax.ShapeDtypeStruct((_B, 128), f32),
    )(t10p, w0p, b0p, w1p, b1p, w2p, b2p)[:, :1]

    s = pl.pallas_call(
        _sum_body,
        grid=(_NCH,),
        in_specs=[
            pl.BlockSpec((_B, _C), lambda j: (0, j)),
            pl.BlockSpec((_B, 1), lambda j: (0, 0)),
            pl.BlockSpec((_B, 1), lambda j: (0, 0)),
        ],
        out_specs=pl.BlockSpec((_B, 128), lambda j: (0, 0)),
        out_shape=jax.ShapeDtypeStruct((_B, 128), f32),
        compiler_params=seq,
    )(logits, m, inv_t)[:, :1]

    probs = pl.pallas_call(
        _scale_body,
        grid=(_NCH,),
        in_specs=[
            pl.BlockSpec((_B, _C), lambda j: (0, j)),
            pl.BlockSpec((_B, 1), lambda j: (0, 0)),
            pl.BlockSpec((_B, 1), lambda j: (0, 0)),
            pl.BlockSpec((_B, 1), lambda j: (0, 0)),
        ],
        out_specs=pl.BlockSpec((_B, _C), lambda j: (0, j)),
        out_shape=jax.ShapeDtypeStruct((_B, _V), f32),
        compiler_params=seq,
    )(logits, m, inv_t, s)

    return probs
